# Initial kernel scaffold; baseline (speedup 1.0000x reference)
import functools

import jax
import jax.numpy as jnp
from jax import lax
from jax.experimental import pallas as pl
from jax.experimental.pallas import tpu as pltpu
from jax.experimental.pallas import tpu_sc as plsc

N = 10000
E = 320000
D = 128
OUT = 128
CAT = D * 2 * 3

NC = 2
NS = 16
LANES = 16

NPAD = 10240
ROWS_PER_TILE = NPAD // NS
CHUNK = 128
NCH = 157
EPAD = NS * NCH * CHUNK
DUMMY = NPAD - 1

_mesh = functools.partial(
    plsc.VectorSubcoreMesh,
    core_axis_name="c", subcore_axis_name="s",
    num_cores=NC, num_subcores=NS,
)


def _sc_degrees(src_r, dst_r):
    @functools.partial(
        pl.kernel,
        out_type=(jax.ShapeDtypeStruct((NPAD,), jnp.float32),
                  jax.ShapeDtypeStruct((NPAD,), jnp.float32)),
        mesh=_mesh(),
        scratch_types=[
            pltpu.VMEM((NCH, CHUNK), jnp.int32),
            pltpu.VMEM((CHUNK,), jnp.float32),
            pltpu.VMEM((ROWS_PER_TILE,), jnp.float32),
        ],
    )
    def k(src_hbm, dst_hbm, dego_hbm, degi_hbm, idx_v, ones_v, zer_v):
        c = lax.axis_index("c")
        s = lax.axis_index("s")

        def fill(ref, n, val):
            def body(i, _):
                ref[pl.ds(i * LANES, LANES)] = jnp.full((LANES,), val, jnp.float32)
                return 0
            lax.fori_loop(0, n // LANES, body, 0)

        fill(ones_v, CHUNK, 1.0)
        fill(zer_v, ROWS_PER_TILE, 0.0)

        def side(idx_hbm, out_hbm, acc_sh):
            pltpu.sync_copy(idx_hbm.at[s], idx_v)
            pltpu.sync_copy(zer_v, acc_sh.at[pl.ds(s * ROWS_PER_TILE, ROWS_PER_TILE)])
            plsc.subcore_barrier()

            def body(j, _):
                pltpu.sync_copy(ones_v, acc_sh.at[idx_v.at[j]], add=True)
                return 0
            lax.fori_loop(0, NCH, body, 0)
            plsc.subcore_barrier()
            sl = pl.ds(s * ROWS_PER_TILE, ROWS_PER_TILE)
            pltpu.sync_copy(acc_sh.at[sl], out_hbm.at[sl])

        def scoped(acc_sh):
            @pl.when(c == 0)
            def _():
                side(src_hbm, dego_hbm, acc_sh)

            @pl.when(c == 1)
            def _():
                side(dst_hbm, degi_hbm, acc_sh)

        pl.run_scoped(scoped, pltpu.VMEM_SHARED((NPAD,), jnp.float32))

    return k(src_r, dst_r)


def _sc_hop(ta, tb, src_r, dst_r):
    @functools.partial(
        pl.kernel,
        out_type=(jax.ShapeDtypeStruct((NPAD, D), jnp.float32),
                  jax.ShapeDtypeStruct((NPAD, D), jnp.float32)),
        mesh=_mesh(),
        scratch_types=[
            pltpu.VMEM((NCH, CHUNK), jnp.int32),
            pltpu.VMEM((NCH, CHUNK), jnp.int32),
            pltpu.VMEM((CHUNK, D), jnp.float32),
            pltpu.SemaphoreType.DMA,
        ],
    )
    def k(ta_hbm, tb_hbm, src_hbm, dst_hbm, outf_hbm, outr_hbm,
          idxg_v, idxs_v, rows_v, sem):
        c = lax.axis_index("c")
        s = lax.axis_index("s")

        def side(table, out, gslab, sslab, acc_sh):
            pltpu.sync_copy(gslab.at[s], idxg_v)
            pltpu.sync_copy(sslab.at[s], idxs_v)

            def zbody(i, _):
                for jj in range(D // LANES):
                    rows_v[i, pl.ds(jj * LANES, LANES)] = jnp.zeros((LANES,), jnp.float32)
                return 0
            lax.fori_loop(0, CHUNK, zbody, 0)
            for r in range(ROWS_PER_TILE // CHUNK):
                pltpu.sync_copy(
                    rows_v, acc_sh.at[pl.ds(s * ROWS_PER_TILE + r * CHUNK, CHUNK)])
            plsc.subcore_barrier()

            def body(j, _):
                pltpu.async_copy(table.at[idxg_v.at[j]], rows_v, sem).wait()
                pltpu.sync_copy(rows_v, acc_sh.at[idxs_v.at[j]], add=True)
                return 0
            lax.fori_loop(0, NCH, body, 0)
            plsc.subcore_barrier()
            sl = pl.ds(s * ROWS_PER_TILE, ROWS_PER_TILE)
            pltpu.sync_copy(acc_sh.at[sl], out.at[sl])

        def scoped(acc_sh):
            @pl.when(c == 0)
            def _():
                side(ta_hbm, outf_hbm, src_hbm, dst_hbm, acc_sh)

            @pl.when(c == 1)
            def _():
                side(tb_hbm, outr_hbm, dst_hbm, src_hbm, acc_sh)

        pl.run_scoped(scoped, pltpu.VMEM_SHARED((NPAD, D), jnp.float32))

    return k(ta, tb, src_r, dst_r)


_BLK = 512


def _row_spec():
    return pl.BlockSpec((_BLK, D), lambda i: (i, 0))


def _deg_spec():
    return pl.BlockSpec((_BLK, 1), lambda i: (i, 0))


def _tc_scale_pre(x_pad, dego, degi):
    def body(x_ref, do_ref, di_ref, ta_ref, tb_ref):
        a = lax.rsqrt(jnp.maximum(do_ref[...], 1.0))
        b = lax.rsqrt(jnp.maximum(di_ref[...], 1.0))
        x = x_ref[...]
        ta_ref[...] = x * a
        tb_ref[...] = x * b

    return pl.pallas_call(
        body,
        grid=(NPAD // _BLK,),
        in_specs=[_row_spec(), _deg_spec(), _deg_spec()],
        out_specs=[_row_spec(), _row_spec()],
        out_shape=[jax.ShapeDtypeStruct((NPAD, D), jnp.float32)] * 2,
    )(x_pad, dego, degi)


def _tc_scale_mid(raw1, raw1r, dego, degi):
    def body(r1_ref, r1r_ref, do_ref, di_ref, t1_ref, t1r_ref):
        a = lax.rsqrt(jnp.maximum(do_ref[...], 1.0))
        b = lax.rsqrt(jnp.maximum(di_ref[...], 1.0))
        ab = a * b
        t1_ref[...] = r1_ref[...] * ab
        t1r_ref[...] = r1r_ref[...] * ab

    return pl.pallas_call(
        body,
        grid=(NPAD // _BLK,),
        in_specs=[_row_spec(), _row_spec(), _deg_spec(), _deg_spec()],
        out_specs=[_row_spec(), _row_spec()],
        out_shape=[jax.ShapeDtypeStruct((NPAD, D), jnp.float32)] * 2,
    )(raw1, raw1r, dego, degi)


def _tc_final(x_pad, raw1, raw2, raw1r, raw2r, dego, degi, Wt, bias):
    def body(x_ref, r1_ref, r2_ref, r1r_ref, r2r_ref, do_ref, di_ref,
             wt_ref, b_ref, o_ref):
        a = lax.rsqrt(jnp.maximum(do_ref[...], 1.0))
        b = lax.rsqrt(jnp.maximum(di_ref[...], 1.0))
        wt = wt_ref[...]
        f32 = jnp.float32
        acc = jnp.dot(x_ref[...], wt[0:D] + wt[3 * D:4 * D],
                      preferred_element_type=f32)
        acc += jnp.dot(r1_ref[...] * b, wt[D:2 * D], preferred_element_type=f32)
        acc += jnp.dot(r2_ref[...] * b, wt[2 * D:3 * D], preferred_element_type=f32)
        acc += jnp.dot(r1r_ref[...] * a, wt[4 * D:5 * D], preferred_element_type=f32)
        acc += jnp.dot(r2r_ref[...] * a, wt[5 * D:6 * D], preferred_element_type=f32)
        o_ref[...] = acc + b_ref[...]

    return pl.pallas_call(
        body,
        grid=(NPAD // _BLK,),
        in_specs=[_row_spec(), _row_spec(), _row_spec(), _row_spec(), _row_spec(),
                  _deg_spec(), _deg_spec(),
                  pl.BlockSpec((CAT, OUT), lambda i: (0, 0)),
                  pl.BlockSpec((1, OUT), lambda i: (0, 0))],
        out_specs=_row_spec(),
        out_shape=jax.ShapeDtypeStruct((NPAD, D), jnp.float32),
    )(x_pad, raw1, raw2, raw1r, raw2r, dego, degi, Wt, bias)


def kernel(feature, edge_index, W, b):
    src = edge_index[0]
    dst = edge_index[1]
    pad = jnp.full((EPAD - E,), DUMMY, jnp.int32)
    src_r = jnp.concatenate([src, pad]).reshape(NS, NCH, CHUNK)
    dst_r = jnp.concatenate([dst, pad]).reshape(NS, NCH, CHUNK)
    x_pad = jnp.pad(feature, ((0, NPAD - N), (0, 0)))

    dego, degi = _sc_degrees(src_r, dst_r)
    dego = dego.reshape(NPAD, 1)
    degi = degi.reshape(NPAD, 1)

    ta, tb = _tc_scale_pre(x_pad, dego, degi)
    raw1, raw1r = _sc_hop(ta, tb, src_r, dst_r)
    t1, t1r = _tc_scale_mid(raw1, raw1r, dego, degi)
    raw2, raw2r = _sc_hop(t1, t1r, src_r, dst_r)

    out_pad = _tc_final(x_pad, raw1, raw2, raw1r, raw2r, dego, degi,
                        W.T, b.reshape(1, OUT))
    return out_pad[:N]


# speedup vs baseline: 6.3085x; 6.3085x over previous
import functools

import jax
import jax.numpy as jnp
from jax import lax
from jax.experimental import pallas as pl
from jax.experimental.pallas import tpu as pltpu
from jax.experimental.pallas import tpu_sc as plsc

N = 10000
E = 320000
D = 128
OUT = 128
CAT = D * 2 * 3

NC = 2
NS = 16
LANES = 16

NPAD = 10240
ROWS_PER_TILE = NPAD // NS
CHUNK = 128
NCH = 160
IB = 16
NB = NCH // IB
EPAD = NS * NCH * CHUNK
DUMMY = NPAD - 1

_mesh = functools.partial(
    plsc.VectorSubcoreMesh,
    core_axis_name="c", subcore_axis_name="s",
    num_cores=NC, num_subcores=NS,
)


def _sc_degrees(src_r, dst_r):
    @functools.partial(
        pl.kernel,
        out_type=(jax.ShapeDtypeStruct((NPAD,), jnp.float32),
                  jax.ShapeDtypeStruct((NPAD,), jnp.float32)),
        mesh=_mesh(),
        scratch_types=[
            pltpu.VMEM((NCH, CHUNK), jnp.int32),
            pltpu.VMEM((CHUNK,), jnp.float32),
            pltpu.VMEM((ROWS_PER_TILE,), jnp.float32),
            pltpu.VMEM_SHARED((NPAD,), jnp.float32),
        ],
    )
    def k(src_hbm, dst_hbm, dego_hbm, degi_hbm, idx_v, ones_v, zer_v, acc_sh):
        c = lax.axis_index("c")
        s = lax.axis_index("s")

        def fill(ref, n, val):
            def body(i, _):
                ref[pl.ds(i * LANES, LANES)] = jnp.full((LANES,), val, jnp.float32)
                return 0
            lax.fori_loop(0, n // LANES, body, 0)

        fill(ones_v, CHUNK, 1.0)
        fill(zer_v, ROWS_PER_TILE, 0.0)

        def side(idx_hbm, out_hbm):
            pltpu.sync_copy(idx_hbm.at[s], idx_v)
            pltpu.sync_copy(zer_v, acc_sh.at[pl.ds(s * ROWS_PER_TILE, ROWS_PER_TILE)])
            plsc.subcore_barrier()

            def body(j, _):
                pltpu.sync_copy(ones_v, acc_sh.at[idx_v.at[j]], add=True)
                return 0
            lax.fori_loop(0, NCH, body, 0)
            plsc.subcore_barrier()
            sl = pl.ds(s * ROWS_PER_TILE, ROWS_PER_TILE)
            pltpu.sync_copy(acc_sh.at[sl], out_hbm.at[sl])

        @pl.when(c == 0)
        def _():
            side(src_hbm, dego_hbm)

        @pl.when(c == 1)
        def _():
            side(dst_hbm, degi_hbm)

    return k(src_r, dst_r)


def _sc_hop(ta, tb, src_r, dst_r):
    @functools.partial(
        pl.kernel,
        out_type=(jax.ShapeDtypeStruct((NPAD, D), jnp.float32),
                  jax.ShapeDtypeStruct((NPAD, D), jnp.float32)),
        mesh=_mesh(),
        scratch_types=[
            pltpu.VMEM((IB, CHUNK), jnp.int32),
            pltpu.VMEM((IB, CHUNK), jnp.int32),
            pltpu.VMEM((CHUNK, D), jnp.float32),
            pltpu.SemaphoreType.DMA,
            pltpu.VMEM_SHARED((NPAD, D), jnp.float32),
        ],
    )
    def k(ta_hbm, tb_hbm, src_hbm, dst_hbm, outf_hbm, outr_hbm,
          idxg_v, idxs_v, rows_v, sem, acc_sh):
        c = lax.axis_index("c")
        s = lax.axis_index("s")

        def side(table, out, gslab, sslab):
            def zbody(i, _):
                for jj in range(D // LANES):
                    rows_v[i, pl.ds(jj * LANES, LANES)] = jnp.zeros((LANES,), jnp.float32)
                return 0
            lax.fori_loop(0, CHUNK, zbody, 0)
            for r in range(ROWS_PER_TILE // CHUNK):
                pltpu.sync_copy(
                    rows_v, acc_sh.at[pl.ds(s * ROWS_PER_TILE + r * CHUNK, CHUNK)])
            plsc.subcore_barrier()

            def blk(nb, _):
                pltpu.sync_copy(gslab.at[s, pl.ds(nb * IB, IB)], idxg_v)
                pltpu.sync_copy(sslab.at[s, pl.ds(nb * IB, IB)], idxs_v)

                def body(j, _):
                    pltpu.async_copy(table.at[idxg_v.at[j]], rows_v, sem).wait()
                    pltpu.sync_copy(rows_v, acc_sh.at[idxs_v.at[j]], add=True)
                    return 0
                lax.fori_loop(0, IB, body, 0)
                return 0
            lax.fori_loop(0, NB, blk, 0)
            plsc.subcore_barrier()
            sl = pl.ds(s * ROWS_PER_TILE, ROWS_PER_TILE)
            pltpu.sync_copy(acc_sh.at[sl], out.at[sl])

        @pl.when(c == 0)
        def _():
            side(ta_hbm, outf_hbm, src_hbm, dst_hbm)

        @pl.when(c == 1)
        def _():
            side(tb_hbm, outr_hbm, dst_hbm, src_hbm)

    return k(ta, tb, src_r, dst_r)


_BLK = 512


def _row_spec():
    return pl.BlockSpec((_BLK, D), lambda i: (i, 0))


def _deg_spec():
    return pl.BlockSpec((_BLK, 1), lambda i: (i, 0))


def _tc_scale_pre(x_pad, dego, degi):
    def body(x_ref, do_ref, di_ref, ta_ref, tb_ref):
        a = lax.rsqrt(jnp.maximum(do_ref[...], 1.0))
        b = lax.rsqrt(jnp.maximum(di_ref[...], 1.0))
        x = x_ref[...]
        ta_ref[...] = x * a
        tb_ref[...] = x * b

    return pl.pallas_call(
        body,
        grid=(NPAD // _BLK,),
        in_specs=[_row_spec(), _deg_spec(), _deg_spec()],
        out_specs=[_row_spec(), _row_spec()],
        out_shape=[jax.ShapeDtypeStruct((NPAD, D), jnp.float32)] * 2,
    )(x_pad, dego, degi)


def _tc_scale_mid(raw1, raw1r, dego, degi):
    def body(r1_ref, r1r_ref, do_ref, di_ref, t1_ref, t1r_ref):
        a = lax.rsqrt(jnp.maximum(do_ref[...], 1.0))
        b = lax.rsqrt(jnp.maximum(di_ref[...], 1.0))
        ab = a * b
        t1_ref[...] = r1_ref[...] * ab
        t1r_ref[...] = r1r_ref[...] * ab

    return pl.pallas_call(
        body,
        grid=(NPAD // _BLK,),
        in_specs=[_row_spec(), _row_spec(), _deg_spec(), _deg_spec()],
        out_specs=[_row_spec(), _row_spec()],
        out_shape=[jax.ShapeDtypeStruct((NPAD, D), jnp.float32)] * 2,
    )(raw1, raw1r, dego, degi)


def _tc_final(x_pad, raw1, raw2, raw1r, raw2r, dego, degi, Wt, bias):
    def body(x_ref, r1_ref, r2_ref, r1r_ref, r2r_ref, do_ref, di_ref,
             wt_ref, b_ref, o_ref):
        a = lax.rsqrt(jnp.maximum(do_ref[...], 1.0))
        b = lax.rsqrt(jnp.maximum(di_ref[...], 1.0))
        wt = wt_ref[...]
        f32 = jnp.float32
        acc = jnp.dot(x_ref[...], wt[0:D] + wt[3 * D:4 * D],
                      preferred_element_type=f32)
        acc += jnp.dot(r1_ref[...] * b, wt[D:2 * D], preferred_element_type=f32)
        acc += jnp.dot(r2_ref[...] * b, wt[2 * D:3 * D], preferred_element_type=f32)
        acc += jnp.dot(r1r_ref[...] * a, wt[4 * D:5 * D], preferred_element_type=f32)
        acc += jnp.dot(r2r_ref[...] * a, wt[5 * D:6 * D], preferred_element_type=f32)
        o_ref[...] = acc + b_ref[...]

    return pl.pallas_call(
        body,
        grid=(NPAD // _BLK,),
        in_specs=[_row_spec(), _row_spec(), _row_spec(), _row_spec(), _row_spec(),
                  _deg_spec(), _deg_spec(),
                  pl.BlockSpec((CAT, OUT), lambda i: (0, 0)),
                  pl.BlockSpec((1, OUT), lambda i: (0, 0))],
        out_specs=_row_spec(),
        out_shape=jax.ShapeDtypeStruct((NPAD, D), jnp.float32),
    )(x_pad, raw1, raw2, raw1r, raw2r, dego, degi, Wt, bias)


def kernel(feature, edge_index, W, b):
    src = edge_index[0]
    dst = edge_index[1]
    pad = jnp.full((EPAD - E,), DUMMY, jnp.int32)
    src_r = jnp.concatenate([src, pad]).reshape(NS, NCH, CHUNK)
    dst_r = jnp.concatenate([dst, pad]).reshape(NS, NCH, CHUNK)
    x_pad = jnp.pad(feature, ((0, NPAD - N), (0, 0)))

    dego, degi = _sc_degrees(src_r, dst_r)
    dego = dego.reshape(NPAD, 1)
    degi = degi.reshape(NPAD, 1)

    ta, tb = _tc_scale_pre(x_pad, dego, degi)
    raw1, raw1r = _sc_hop(ta, tb, src_r, dst_r)
    t1, t1r = _tc_scale_mid(raw1, raw1r, dego, degi)
    raw2, raw2r = _sc_hop(t1, t1r, src_r, dst_r)

    out_pad = _tc_final(x_pad, raw1, raw2, raw1r, raw2r, dego, degi,
                        W.T, b.reshape(1, OUT))
    return out_pad[:N]


# speedup vs baseline: 7.4837x; 1.1863x over previous
import functools

import jax
import jax.numpy as jnp
from jax import lax
from jax.experimental import pallas as pl
from jax.experimental.pallas import tpu as pltpu
from jax.experimental.pallas import tpu_sc as plsc

N = 10000
E = 320000
D = 128
OUT = 128
CAT = D * 2 * 3

NC = 2
NS = 16
LANES = 16

NPAD = 10240
ROWS_PER_TILE = NPAD // NS
CHUNK = 128
NCH = 160
IB = 16
NB = NCH // IB
EPAD = NS * NCH * CHUNK
DUMMY = NPAD - 1

_mesh = functools.partial(
    plsc.VectorSubcoreMesh,
    core_axis_name="c", subcore_axis_name="s",
    num_cores=NC, num_subcores=NS,
)


def _sc_degrees(src_r, dst_r):
    @functools.partial(
        pl.kernel,
        out_type=(jax.ShapeDtypeStruct((NPAD,), jnp.float32),
                  jax.ShapeDtypeStruct((NPAD,), jnp.float32)),
        mesh=_mesh(),
        scratch_types=[
            pltpu.VMEM((NCH, CHUNK), jnp.int32),
            pltpu.VMEM((CHUNK,), jnp.float32),
            pltpu.VMEM((ROWS_PER_TILE,), jnp.float32),
            pltpu.VMEM_SHARED((NPAD,), jnp.float32),
        ],
    )
    def k(src_hbm, dst_hbm, dego_hbm, degi_hbm, idx_v, ones_v, zer_v, acc_sh):
        c = lax.axis_index("c")
        s = lax.axis_index("s")

        def fill(ref, n, val):
            def body(i, _):
                ref[pl.ds(i * LANES, LANES)] = jnp.full((LANES,), val, jnp.float32)
                return 0
            lax.fori_loop(0, n // LANES, body, 0)

        fill(ones_v, CHUNK, 1.0)
        fill(zer_v, ROWS_PER_TILE, 0.0)

        def side(idx_hbm, out_hbm):
            pltpu.sync_copy(idx_hbm.at[s], idx_v)
            pltpu.sync_copy(zer_v, acc_sh.at[pl.ds(s * ROWS_PER_TILE, ROWS_PER_TILE)])
            plsc.subcore_barrier()

            def body(j, _):
                pltpu.sync_copy(ones_v, acc_sh.at[idx_v.at[j]], add=True)
                return 0
            lax.fori_loop(0, NCH, body, 0)
            plsc.subcore_barrier()
            sl = pl.ds(s * ROWS_PER_TILE, ROWS_PER_TILE)
            pltpu.sync_copy(acc_sh.at[sl], out_hbm.at[sl])

        @pl.when(c == 0)
        def _():
            side(src_hbm, dego_hbm)

        @pl.when(c == 1)
        def _():
            side(dst_hbm, degi_hbm)

    return k(src_r, dst_r)


def _sc_hop(ta, tb, src_r, dst_r):
    @functools.partial(
        pl.kernel,
        out_type=(jax.ShapeDtypeStruct((NPAD, D), jnp.float32),
                  jax.ShapeDtypeStruct((NPAD, D), jnp.float32)),
        mesh=_mesh(),
        scratch_types=[
            pltpu.VMEM((IB, CHUNK), jnp.int32),
            pltpu.VMEM((IB, CHUNK), jnp.int32),
            pltpu.VMEM((CHUNK, D), jnp.float32),
            pltpu.VMEM((CHUNK, D), jnp.float32),
            pltpu.SemaphoreType.DMA,
            pltpu.SemaphoreType.DMA,
            pltpu.VMEM_SHARED((NPAD, D), jnp.float32),
        ],
    )
    def k(ta_hbm, tb_hbm, src_hbm, dst_hbm, outf_hbm, outr_hbm,
          idxg_v, idxs_v, rows_a, rows_b, sem_a, sem_b, acc_sh):
        c = lax.axis_index("c")
        s = lax.axis_index("s")

        def side(table, out, gslab, sslab):
            def zbody(i, _):
                for jj in range(D // LANES):
                    rows_a[i, pl.ds(jj * LANES, LANES)] = jnp.zeros((LANES,), jnp.float32)
                return 0
            lax.fori_loop(0, CHUNK, zbody, 0)
            for r in range(ROWS_PER_TILE // CHUNK):
                pltpu.sync_copy(
                    rows_a, acc_sh.at[pl.ds(s * ROWS_PER_TILE + r * CHUNK, CHUNK)])
            plsc.subcore_barrier()

            def gath(j, buf, sem):
                pltpu.async_copy(table.at[idxg_v.at[j]], buf, sem)

            def scat(j, buf, sem):
                pltpu.make_async_copy(table.at[idxg_v.at[j]], buf, sem).wait()
                pltpu.sync_copy(buf, acc_sh.at[idxs_v.at[j]], add=True)

            def blk(nb, _):
                pltpu.sync_copy(gslab.at[s, pl.ds(nb * IB, IB)], idxg_v)
                pltpu.sync_copy(sslab.at[s, pl.ds(nb * IB, IB)], idxs_v)
                gath(0, rows_a, sem_a)

                def pair(p, _):
                    gath(2 * p + 1, rows_b, sem_b)
                    scat(2 * p, rows_a, sem_a)
                    gath(2 * p + 2, rows_a, sem_a)
                    scat(2 * p + 1, rows_b, sem_b)
                    return 0
                lax.fori_loop(0, IB // 2 - 1, pair, 0)
                gath(IB - 1, rows_b, sem_b)
                scat(IB - 2, rows_a, sem_a)
                scat(IB - 1, rows_b, sem_b)
                return 0
            lax.fori_loop(0, NB, blk, 0)
            plsc.subcore_barrier()
            sl = pl.ds(s * ROWS_PER_TILE, ROWS_PER_TILE)
            pltpu.sync_copy(acc_sh.at[sl], out.at[sl])

        @pl.when(c == 0)
        def _():
            side(ta_hbm, outf_hbm, src_hbm, dst_hbm)

        @pl.when(c == 1)
        def _():
            side(tb_hbm, outr_hbm, dst_hbm, src_hbm)

    return k(ta, tb, src_r, dst_r)


_BLK = 512


def _row_spec():
    return pl.BlockSpec((_BLK, D), lambda i: (i, 0))


def _deg_spec():
    return pl.BlockSpec((_BLK, 1), lambda i: (i, 0))


def _tc_scale_pre(x_pad, dego, degi):
    def body(x_ref, do_ref, di_ref, ta_ref, tb_ref):
        a = lax.rsqrt(jnp.maximum(do_ref[...], 1.0))
        b = lax.rsqrt(jnp.maximum(di_ref[...], 1.0))
        x = x_ref[...]
        ta_ref[...] = x * a
        tb_ref[...] = x * b

    return pl.pallas_call(
        body,
        grid=(NPAD // _BLK,),
        in_specs=[_row_spec(), _deg_spec(), _deg_spec()],
        out_specs=[_row_spec(), _row_spec()],
        out_shape=[jax.ShapeDtypeStruct((NPAD, D), jnp.float32)] * 2,
    )(x_pad, dego, degi)


def _tc_scale_mid(raw1, raw1r, dego, degi):
    def body(r1_ref, r1r_ref, do_ref, di_ref, t1_ref, t1r_ref):
        a = lax.rsqrt(jnp.maximum(do_ref[...], 1.0))
        b = lax.rsqrt(jnp.maximum(di_ref[...], 1.0))
        ab = a * b
        t1_ref[...] = r1_ref[...] * ab
        t1r_ref[...] = r1r_ref[...] * ab

    return pl.pallas_call(
        body,
        grid=(NPAD // _BLK,),
        in_specs=[_row_spec(), _row_spec(), _deg_spec(), _deg_spec()],
        out_specs=[_row_spec(), _row_spec()],
        out_shape=[jax.ShapeDtypeStruct((NPAD, D), jnp.float32)] * 2,
    )(raw1, raw1r, dego, degi)


def _tc_final(x_pad, raw1, raw2, raw1r, raw2r, dego, degi, Wt, bias):
    def body(x_ref, r1_ref, r2_ref, r1r_ref, r2r_ref, do_ref, di_ref,
             wt_ref, b_ref, o_ref):
        a = lax.rsqrt(jnp.maximum(do_ref[...], 1.0))
        b = lax.rsqrt(jnp.maximum(di_ref[...], 1.0))
        wt = wt_ref[...]
        f32 = jnp.float32
        acc = jnp.dot(x_ref[...], wt[0:D] + wt[3 * D:4 * D],
                      preferred_element_type=f32)
        acc += jnp.dot(r1_ref[...] * b, wt[D:2 * D], preferred_element_type=f32)
        acc += jnp.dot(r2_ref[...] * b, wt[2 * D:3 * D], preferred_element_type=f32)
        acc += jnp.dot(r1r_ref[...] * a, wt[4 * D:5 * D], preferred_element_type=f32)
        acc += jnp.dot(r2r_ref[...] * a, wt[5 * D:6 * D], preferred_element_type=f32)
        o_ref[...] = acc + b_ref[...]

    return pl.pallas_call(
        body,
        grid=(NPAD // _BLK,),
        in_specs=[_row_spec(), _row_spec(), _row_spec(), _row_spec(), _row_spec(),
                  _deg_spec(), _deg_spec(),
                  pl.BlockSpec((CAT, OUT), lambda i: (0, 0)),
                  pl.BlockSpec((1, OUT), lambda i: (0, 0))],
        out_specs=_row_spec(),
        out_shape=jax.ShapeDtypeStruct((NPAD, D), jnp.float32),
    )(x_pad, raw1, raw2, raw1r, raw2r, dego, degi, Wt, bias)


def kernel(feature, edge_index, W, b):
    src = edge_index[0]
    dst = edge_index[1]
    pad = jnp.full((EPAD - E,), DUMMY, jnp.int32)
    src_r = jnp.concatenate([src, pad]).reshape(NS, NCH, CHUNK)
    dst_r = jnp.concatenate([dst, pad]).reshape(NS, NCH, CHUNK)
    x_pad = jnp.pad(feature, ((0, NPAD - N), (0, 0)))

    dego, degi = _sc_degrees(src_r, dst_r)
    dego = dego.reshape(NPAD, 1)
    degi = degi.reshape(NPAD, 1)

    ta, tb = _tc_scale_pre(x_pad, dego, degi)
    raw1, raw1r = _sc_hop(ta, tb, src_r, dst_r)
    t1, t1r = _tc_scale_mid(raw1, raw1r, dego, degi)
    raw2, raw2r = _sc_hop(t1, t1r, src_r, dst_r)

    out_pad = _tc_final(x_pad, raw1, raw2, raw1r, raw2r, dego, degi,
                        W.T, b.reshape(1, OUT))
    return out_pad[:N]
